# Initial kernel scaffold; baseline (speedup 1.0000x reference)
import jax
import jax.numpy as jnp
from jax.experimental import pallas as pl


def kernel(query_pre_summary, bank_summaries, bank_templates, W, b, gamma, beta, gate_logit):
    raise NotImplementedError("write your pallas kernel here")


# speedup vs baseline: 1.5512x; 1.5512x over previous
import functools

import jax
import jax.numpy as jnp
from jax import lax
from jax.experimental import pallas as pl
from jax.experimental.pallas import tpu as pltpu

B, C, N, NT, K = 256, 256, 4096, 32, 8


def _topk_body(q_ref, s_ref, g_ref, idx_ref, gate_ref):
    q = q_ref[...]
    s = s_ref[...]
    qn = q / jnp.maximum(jnp.sqrt(jnp.sum(q * q, axis=1, keepdims=True)), 1e-12)
    sn = s / jnp.maximum(jnp.sqrt(jnp.sum(s * s, axis=1, keepdims=True)), 1e-12)
    sims = lax.dot_general(
        qn.astype(jnp.bfloat16), sn.astype(jnp.bfloat16),
        dimension_numbers=(((1,), (1,)), ((), ())),
        preferred_element_type=jnp.float32,
    )
    iota = lax.broadcasted_iota(jnp.int32, (B, N), 1)
    neg = jnp.float32(-jnp.inf)
    cols = []
    for _ in range(K):
        m = jnp.max(sims, axis=1, keepdims=True)
        idxk = jnp.min(jnp.where(sims >= m, iota, N), axis=1)
        cols.append(idxk)
        sims = jnp.where(iota == idxk[:, None], neg, sims)
    idx_ref[...] = jnp.stack(cols, axis=1)
    gate = jax.nn.sigmoid(g_ref[0, 0])
    gate_ref[...] = jnp.full((B, 1), gate, jnp.float32)


def _proj_body(idx_ref, t0, t1, t2, t3, t4, t5, t6, t7,
               w_ref, b_ref, gm_ref, bt_ref, gate_ref, out_ref):
    x = jnp.concatenate(
        [t0[0], t1[0], t2[0], t3[0], t4[0], t5[0], t6[0], t7[0]], axis=0
    )
    h = lax.dot_general(
        x, w_ref[...],
        dimension_numbers=(((1,), (1,)), ((), ())),
        preferred_element_type=jnp.float32,
    ) + b_ref[...]
    mu = jnp.mean(h, axis=1, keepdims=True)
    d = h - mu
    var = jnp.mean(d * d, axis=1, keepdims=True)
    hn = d * lax.rsqrt(var + 1e-5) * gm_ref[...] + bt_ref[...]
    out_ref[0] = hn * gate_ref[0, 0]


def kernel(query_pre_summary, bank_summaries, bank_templates, W, b, gamma, beta, gate_logit):
    g_arr = jnp.reshape(gate_logit.astype(jnp.float32), (1, 1))
    idx, gate_b = pl.pallas_call(
        _topk_body,
        out_shape=[
            jax.ShapeDtypeStruct((B, K), jnp.int32),
            jax.ShapeDtypeStruct((B, 1), jnp.float32),
        ],
    )(query_pre_summary, bank_summaries, g_arr)

    def t_map(k):
        def m(bb, idx_ref):
            return (idx_ref[bb, k], 0, 0)
        return m

    grid_spec = pltpu.PrefetchScalarGridSpec(
        num_scalar_prefetch=1,
        grid=(B,),
        in_specs=(
            [pl.BlockSpec((1, NT, C), t_map(k)) for k in range(K)]
            + [
                pl.BlockSpec((C, C), lambda bb, idx_ref: (0, 0)),
                pl.BlockSpec((1, C), lambda bb, idx_ref: (0, 0)),
                pl.BlockSpec((1, C), lambda bb, idx_ref: (0, 0)),
                pl.BlockSpec((1, C), lambda bb, idx_ref: (0, 0)),
                pl.BlockSpec((B, 1), lambda bb, idx_ref: (0, 0)),
            ]
        ),
        out_specs=pl.BlockSpec((1, K * NT, C), lambda bb, idx_ref: (bb, 0, 0)),
    )
    retrieved = pl.pallas_call(
        _proj_body,
        grid_spec=grid_spec,
        out_shape=jax.ShapeDtypeStruct((B, K * NT, C), jnp.float32),
    )(
        idx,
        bank_templates, bank_templates, bank_templates, bank_templates,
        bank_templates, bank_templates, bank_templates, bank_templates,
        W,
        jnp.reshape(b, (1, C)),
        jnp.reshape(gamma, (1, C)),
        jnp.reshape(beta, (1, C)),
        gate_b,
    )
    return retrieved, gate_b
